# baseline (speedup 1.0000x reference)
import functools

import jax
import jax.numpy as jnp
from jax import lax
from jax.experimental import pallas as pl

B = 1024
IN_DIM = 4096
H0 = 1024
H1 = 4096
K = 512
EMB = 512
P = H1 // EMB


def _mm_act_kernel(act, in_bf16, x_ref, w_ref, b_ref, o_ref):
    x = x_ref[...]
    if in_bf16 and x.dtype != jnp.bfloat16:
        x = x.astype(jnp.bfloat16)
    y = jnp.dot(x, w_ref[...], preferred_element_type=jnp.float32)
    y = y + b_ref[...]
    if act == "relu":
        y = jax.nn.relu(y)
    elif act == "tanh":
        y = jnp.tanh(y)
    o_ref[...] = y.astype(o_ref.dtype)


def _mm_act(x, w, b, act, out_dtype=jnp.float32, in_bf16=False, bm=512):
    m, k = x.shape
    n = w.shape[1]
    grid = (m // bm,)
    return pl.pallas_call(
        functools.partial(_mm_act_kernel, act, in_bf16),
        grid=grid,
        in_specs=[
            pl.BlockSpec((bm, k), lambda i: (i, 0)),
            pl.BlockSpec((k, n), lambda i: (0, 0)),
            pl.BlockSpec((1, n), lambda i: (0, 0)),
        ],
        out_specs=pl.BlockSpec((bm, n), lambda i: (i, 0)),
        out_shape=jax.ShapeDtypeStruct((m, n), out_dtype),
    )(x, w, b.reshape(1, n))


def _vq_kernel(zt_ref, c_ref, ct_ref, emb_ref):
    c = c_ref[...]
    c2 = jnp.sum(c * c, axis=0, keepdims=True)
    d = c2 - 2.0 * jnp.dot(zt_ref[...], c, preferred_element_type=jnp.float32)
    mn = jnp.min(d, axis=1, keepdims=True)
    iot = lax.broadcasted_iota(jnp.int32, d.shape, 1)
    idx = jnp.min(jnp.where(d == mn, iot, K), axis=1, keepdims=True)
    oh = (iot == idx).astype(jnp.bfloat16)
    emb_ref[...] = jnp.dot(oh, ct_ref[...].astype(jnp.bfloat16),
                           preferred_element_type=jnp.float32)


def _vq_quantize(zt, codebook, ct, bm=1024):
    n = zt.shape[0]
    grid = (n // bm,)
    return pl.pallas_call(
        _vq_kernel,
        grid=grid,
        in_specs=[
            pl.BlockSpec((bm, EMB), lambda i: (i, 0)),
            pl.BlockSpec((EMB, K), lambda i: (0, 0)),
            pl.BlockSpec((K, EMB), lambda i: (0, 0)),
        ],
        out_specs=pl.BlockSpec((bm, EMB), lambda i: (i, 0)),
        out_shape=jax.ShapeDtypeStruct((n, EMB), jnp.float32),
    )(zt, codebook, ct)


def kernel(x, W1, b1, W2, b2, W3, b3, W4, b4, codebook):
    w3p = (W3.reshape(EMB, P, H0).transpose(1, 0, 2)
           .reshape(H1, H0).astype(jnp.bfloat16))
    w4b = W4.astype(jnp.bfloat16)
    ct = codebook.transpose(1, 0)

    h1 = _mm_act(x, W1, b1, "relu")
    h2 = _mm_act(h1, W2, b2, "none")
    z_e = h2.reshape(B, EMB, P)

    zt = h2.reshape(B, EMB, P).transpose(0, 2, 1).reshape(B * P, EMB)
    halves = []
    H = B // 2
    for h in range(2):
        zth = jax.lax.slice(zt, (h * H * P, 0), ((h + 1) * H * P, EMB))
        embt = _vq_quantize(zth, codebook, ct, bm=1024)
        zf = embt.reshape(H, H1)
        h3 = _mm_act(zf, w3p, b3, "relu", out_dtype=jnp.bfloat16, in_bf16=True)
        rec = _mm_act(h3, w4b, b4, "tanh", in_bf16=True)
        embh = embt.reshape(H, P, EMB).transpose(0, 2, 1)
        halves.append((rec, embh))
    recon = jnp.concatenate([halves[0][0], halves[1][0]], axis=0)
    emb = jnp.concatenate([halves[0][1], halves[1][1]], axis=0)
    return (recon, z_e, emb)


# speedup vs baseline: 1.3391x; 1.3391x over previous
import functools

import jax
import jax.numpy as jnp
from jax import lax
from jax.experimental import pallas as pl

B = 1024
IN_DIM = 4096
H0 = 1024
H1 = 4096
K = 512
EMB = 512
P = H1 // EMB


def _mm_act_kernel(act, in_bf16, x_ref, w_ref, b_ref, o_ref):
    x = x_ref[...]
    if in_bf16 and x.dtype != jnp.bfloat16:
        x = x.astype(jnp.bfloat16)
    y = jnp.dot(x, w_ref[...], preferred_element_type=jnp.float32)
    y = y + b_ref[...]
    if act == "relu":
        y = jax.nn.relu(y)
    elif act == "tanh":
        y = jnp.tanh(y)
    o_ref[...] = y.astype(o_ref.dtype)


def _mm_act(x, w, b, act, out_dtype=jnp.float32, in_bf16=False, bm=512):
    m, k = x.shape
    n = w.shape[1]
    grid = (m // bm,)
    return pl.pallas_call(
        functools.partial(_mm_act_kernel, act, in_bf16),
        grid=grid,
        in_specs=[
            pl.BlockSpec((bm, k), lambda i: (i, 0)),
            pl.BlockSpec((k, n), lambda i: (0, 0)),
            pl.BlockSpec((1, n), lambda i: (0, 0)),
        ],
        out_specs=pl.BlockSpec((bm, n), lambda i: (i, 0)),
        out_shape=jax.ShapeDtypeStruct((m, n), out_dtype),
    )(x, w, b.reshape(1, n))


def _vq_kernel(zt_ref, c_ref, ct_ref, emb_ref):
    c = c_ref[...]
    c2 = jnp.sum(c * c, axis=0, keepdims=True)
    d = c2 - 2.0 * jnp.dot(zt_ref[...], c, preferred_element_type=jnp.float32)
    mn = jnp.min(d, axis=1, keepdims=True)
    iot = lax.broadcasted_iota(jnp.int32, d.shape, 1)
    idx = jnp.min(jnp.where(d == mn, iot, K), axis=1, keepdims=True)
    oh = (iot == idx).astype(jnp.bfloat16)
    emb_ref[...] = jnp.dot(oh, ct_ref[...].astype(jnp.bfloat16),
                           preferred_element_type=jnp.float32)


def _vq_quantize(zt, codebook, ct, bm=1024):
    n = zt.shape[0]
    grid = (n // bm,)
    return pl.pallas_call(
        _vq_kernel,
        grid=grid,
        in_specs=[
            pl.BlockSpec((bm, EMB), lambda i: (i, 0)),
            pl.BlockSpec((EMB, K), lambda i: (0, 0)),
            pl.BlockSpec((K, EMB), lambda i: (0, 0)),
        ],
        out_specs=pl.BlockSpec((bm, EMB), lambda i: (i, 0)),
        out_shape=jax.ShapeDtypeStruct((n, EMB), jnp.float32),
    )(zt, codebook, ct)


def kernel(x, W1, b1, W2, b2, W3, b3, W4, b4, codebook):
    w3p = (W3.reshape(EMB, P, H0).transpose(1, 0, 2)
           .reshape(H1, H0).astype(jnp.bfloat16))
    w4b = W4.astype(jnp.bfloat16)
    ct = codebook.transpose(1, 0)

    h1 = _mm_act(x, W1, b1, "relu")
    h2 = _mm_act(h1, W2, b2, "none")
    z_e = h2.reshape(B, EMB, P)

    zt = h2.reshape(B, EMB, P).transpose(0, 2, 1).reshape(B * P, EMB)
    embt = _vq_quantize(zt, codebook, ct)

    zf = embt.reshape(B, H1)
    h3 = _mm_act(zf, w3p, b3, "relu", out_dtype=jnp.bfloat16, in_bf16=True)
    recon = _mm_act(h3, w4b, b4, "tanh", in_bf16=True)

    emb = embt.reshape(B, P, EMB).transpose(0, 2, 1)
    return (recon, z_e, emb)


# speedup vs baseline: 1.3461x; 1.0052x over previous
import functools

import jax
import jax.numpy as jnp
from jax import lax
from jax.experimental import pallas as pl

B = 1024
IN_DIM = 4096
H0 = 1024
H1 = 4096
K = 512
EMB = 512
P = H1 // EMB


def _mm_act_kernel(act, in_bf16, x_ref, w_ref, b_ref, o_ref):
    x = x_ref[...]
    if in_bf16 and x.dtype != jnp.bfloat16:
        x = x.astype(jnp.bfloat16)
    y = jnp.dot(x, w_ref[...], preferred_element_type=jnp.float32)
    y = y + b_ref[...]
    if act == "relu":
        y = jax.nn.relu(y)
    elif act == "tanh":
        y = jnp.tanh(y)
    o_ref[...] = y.astype(o_ref.dtype)


def _mm_act(x, w, b, act, out_dtype=jnp.float32, in_bf16=False, bm=512):
    m, k = x.shape
    n = w.shape[1]
    grid = (m // bm,)
    return pl.pallas_call(
        functools.partial(_mm_act_kernel, act, in_bf16),
        grid=grid,
        in_specs=[
            pl.BlockSpec((bm, k), lambda i: (i, 0)),
            pl.BlockSpec((k, n), lambda i: (0, 0)),
            pl.BlockSpec((1, n), lambda i: (0, 0)),
        ],
        out_specs=pl.BlockSpec((bm, n), lambda i: (i, 0)),
        out_shape=jax.ShapeDtypeStruct((m, n), out_dtype),
    )(x, w, b.reshape(1, n))


def _vq_kernel(zt_ref, c_ref, ct_ref, emb_ref):
    c = c_ref[...]
    c2 = jnp.sum(c * c, axis=0, keepdims=True)
    d = c2 - 2.0 * jnp.dot(zt_ref[...], c, preferred_element_type=jnp.float32)
    mn = jnp.min(d, axis=1, keepdims=True)
    iot = lax.broadcasted_iota(jnp.int32, d.shape, 1)
    idx = jnp.min(jnp.where(d == mn, iot, K), axis=1, keepdims=True)
    oh = (iot == idx).astype(jnp.bfloat16)
    emb_ref[...] = jnp.dot(oh, ct_ref[...].astype(jnp.bfloat16),
                           preferred_element_type=jnp.float32)


def _vq_quantize(zt, codebook, ct, bm=2048):
    n = zt.shape[0]
    grid = (n // bm,)
    return pl.pallas_call(
        _vq_kernel,
        grid=grid,
        in_specs=[
            pl.BlockSpec((bm, EMB), lambda i: (i, 0)),
            pl.BlockSpec((EMB, K), lambda i: (0, 0)),
            pl.BlockSpec((K, EMB), lambda i: (0, 0)),
        ],
        out_specs=pl.BlockSpec((bm, EMB), lambda i: (i, 0)),
        out_shape=jax.ShapeDtypeStruct((n, EMB), jnp.float32),
    )(zt, codebook, ct)


def kernel(x, W1, b1, W2, b2, W3, b3, W4, b4, codebook):
    w3p = (W3.reshape(EMB, P, H0).transpose(1, 0, 2)
           .reshape(H1, H0).astype(jnp.bfloat16))
    w4b = W4.astype(jnp.bfloat16)
    ct = codebook.transpose(1, 0)

    h1 = _mm_act(x, W1, b1, "relu")
    h2 = _mm_act(h1, W2, b2, "none")
    z_e = h2.reshape(B, EMB, P)

    zt = h2.reshape(B, EMB, P).transpose(0, 2, 1).reshape(B * P, EMB)
    embt = _vq_quantize(zt, codebook, ct)

    zf = embt.reshape(B, H1)
    h3 = _mm_act(zf, w3p, b3, "relu", out_dtype=jnp.bfloat16, in_bf16=True)
    recon = _mm_act(h3, w4b, b4, "tanh", in_bf16=True)

    emb = embt.reshape(B, P, EMB).transpose(0, 2, 1)
    return (recon, z_e, emb)


# speedup vs baseline: 1.3498x; 1.0027x over previous
import functools

import jax
import jax.numpy as jnp
from jax import lax
from jax.experimental import pallas as pl

B = 1024
IN_DIM = 4096
H0 = 1024
H1 = 4096
K = 512
EMB = 512
P = H1 // EMB


def _mm_act_kernel(act, in_bf16, x_ref, w_ref, b_ref, o_ref):
    x = x_ref[...]
    if in_bf16 and x.dtype != jnp.bfloat16:
        x = x.astype(jnp.bfloat16)
    y = jnp.dot(x, w_ref[...], preferred_element_type=jnp.float32)
    y = y + b_ref[...]
    if act == "relu":
        y = jax.nn.relu(y)
    elif act == "tanh":
        y = jnp.tanh(y)
    o_ref[...] = y.astype(o_ref.dtype)


def _mm_act(x, w, b, act, out_dtype=jnp.float32, in_bf16=False, bm=512):
    m, k = x.shape
    n = w.shape[1]
    grid = (m // bm,)
    return pl.pallas_call(
        functools.partial(_mm_act_kernel, act, in_bf16),
        grid=grid,
        in_specs=[
            pl.BlockSpec((bm, k), lambda i: (i, 0)),
            pl.BlockSpec((k, n), lambda i: (0, 0)),
            pl.BlockSpec((1, n), lambda i: (0, 0)),
        ],
        out_specs=pl.BlockSpec((bm, n), lambda i: (i, 0)),
        out_shape=jax.ShapeDtypeStruct((m, n), out_dtype),
    )(x, w, b.reshape(1, n))


def _vq_kernel(zt_ref, c_ref, ct_ref, emb_ref):
    c = c_ref[...]
    c2 = jnp.sum(c * c, axis=0, keepdims=True)
    d = c2 - 2.0 * jnp.dot(zt_ref[...], c, preferred_element_type=jnp.float32)
    mn = jnp.min(d, axis=1, keepdims=True)
    iot = lax.broadcasted_iota(jnp.int32, d.shape, 1)
    idx = jnp.min(jnp.where(d == mn, iot, K), axis=1, keepdims=True)
    oh = (iot == idx).astype(jnp.bfloat16)
    emb_ref[...] = jnp.dot(oh, ct_ref[...].astype(jnp.bfloat16),
                           preferred_element_type=jnp.float32)


def _vq_quantize(zt, codebook, ct, bm=4096):
    n = zt.shape[0]
    grid = (n // bm,)
    return pl.pallas_call(
        _vq_kernel,
        grid=grid,
        in_specs=[
            pl.BlockSpec((bm, EMB), lambda i: (i, 0)),
            pl.BlockSpec((EMB, K), lambda i: (0, 0)),
            pl.BlockSpec((K, EMB), lambda i: (0, 0)),
        ],
        out_specs=pl.BlockSpec((bm, EMB), lambda i: (i, 0)),
        out_shape=jax.ShapeDtypeStruct((n, EMB), jnp.float32),
    )(zt, codebook, ct)


def kernel(x, W1, b1, W2, b2, W3, b3, W4, b4, codebook):
    w3p = (W3.reshape(EMB, P, H0).transpose(1, 0, 2)
           .reshape(H1, H0).astype(jnp.bfloat16))
    w4b = W4.astype(jnp.bfloat16)
    ct = codebook.transpose(1, 0)

    h1 = _mm_act(x, W1, b1, "relu")
    h2 = _mm_act(h1, W2, b2, "none")
    z_e = h2.reshape(B, EMB, P)

    zt = h2.reshape(B, EMB, P).transpose(0, 2, 1).reshape(B * P, EMB)
    embt = _vq_quantize(zt, codebook, ct)

    zf = embt.reshape(B, H1)
    h3 = _mm_act(zf, w3p, b3, "relu", out_dtype=jnp.bfloat16, in_bf16=True)
    recon = _mm_act(h3, w4b, b4, "tanh", in_bf16=True)

    emb = embt.reshape(B, P, EMB).transpose(0, 2, 1)
    return (recon, z_e, emb)
